# baseline (speedup 1.0000x reference)
import functools

import jax
import jax.numpy as jnp
from jax import lax
from jax.experimental import pallas as pl
from jax.experimental.pallas import tpu as pltpu
from jax.experimental.pallas import tpu_sc as plsc

N_NODES = 10000
N_EDGES = 320000
D_FEAT = 128

NC = 2
NS = 16
NW = NC * NS
BLK = 128
EPW_REAL = N_EDGES // NW
NBLK = 80
CH = 16
EPW = NBLK * BLK
PAD_N = EPW - EPW_REAL
PAD_ROWS = 16
ROWS_PER_TILE = 632
ACC_ROWS = NS * ROWS_PER_TILE


def _sc_gather_scatter(feat, idx, zeros):
    mesh = plsc.VectorSubcoreMesh(
        core_axis_name="c", subcore_axis_name="s",
        num_cores=NC, num_subcores=NS)

    @functools.partial(
        pl.kernel,
        out_type=jax.ShapeDtypeStruct((NC, ACC_ROWS, D_FEAT), jnp.float32),
        mesh=mesh,
        scratch_types=[
            pltpu.VMEM((CH, BLK), jnp.int32),
            pltpu.VMEM((CH, BLK), jnp.int32),
            pltpu.VMEM((2, BLK, D_FEAT), jnp.float32),
            pltpu.VMEM_SHARED((ACC_ROWS, D_FEAT), jnp.float32),
            pltpu.SemaphoreType.DMA,
            pltpu.SemaphoreType.DMA,
        ],
    )
    def k(feat_hbm, idx_hbm, zeros_hbm, out_hbm,
          src_v, dst_v, rows_v, accum, sem0, sem1):
        c = lax.axis_index("c")
        s = lax.axis_index("s")
        wid = s * NC + c
        row0 = s * ROWS_PER_TILE

        @pl.when(c == 0)
        def _():
            f0 = lax.min(row0, N_NODES - ROWS_PER_TILE)
            pltpu.sync_copy(feat_hbm.at[pl.ds(f0, ROWS_PER_TILE)],
                            accum.at[pl.ds(f0, ROWS_PER_TILE)])

        @pl.when(c == 1)
        def _():
            pltpu.sync_copy(zeros_hbm,
                            accum.at[pl.ds(row0, ROWS_PER_TILE)])

        plsc.subcore_barrier()

        sems = (sem0, sem1)

        HB = BLK // 2

        def gather_start(j, b):
            pltpu.async_copy(feat_hbm.at[src_v.at[j, pl.ds(0, HB)]],
                             rows_v.at[b, pl.ds(0, HB)], sems[b])
            pltpu.async_copy(feat_hbm.at[src_v.at[j, pl.ds(HB, HB)]],
                             rows_v.at[b, pl.ds(HB, HB)], sems[b])

        def gather_wait(j, b):
            pltpu.make_async_copy(
                feat_hbm.at[src_v.at[j]], rows_v.at[b], sems[b]).wait()

        def chunk_body(ci, carry):
            pltpu.sync_copy(idx_hbm.at[0, wid, pl.ds(ci * CH, CH)], src_v)
            pltpu.sync_copy(idx_hbm.at[1, wid, pl.ds(ci * CH, CH)], dst_v)
            gather_start(0, 0)

            def pair_body(i, carry2):
                g = i * 2
                for b in range(2):
                    j = g + b
                    gather_start((j + 1) % CH, 1 - b)
                    gather_wait(j, b)
                    pltpu.sync_copy(rows_v.at[b], accum.at[dst_v.at[j]],
                                    add=True)
                return carry2

            lax.fori_loop(0, CH // 2, pair_body, 0)
            gather_wait(0, 0)
            return carry

        lax.fori_loop(0, NBLK // CH, chunk_body, 0)

        plsc.subcore_barrier()
        pltpu.sync_copy(accum.at[pl.ds(row0, ROWS_PER_TILE)],
                        out_hbm.at[c, pl.ds(row0, ROWS_PER_TILE)])

    return k(feat, idx, zeros)


def _linear(parts, W, b2):
    BM = 1000

    def mm_body(p0_ref, p1_ref, w_ref, b_ref, o_ref):
        h = p0_ref[0] + p1_ref[0]
        o_ref[...] = jnp.dot(h, w_ref[...],
                             preferred_element_type=jnp.float32) + b_ref[...]

    return pl.pallas_call(
        mm_body,
        grid=(N_NODES // BM,),
        in_specs=[
            pl.BlockSpec((1, BM, D_FEAT), lambda i: (0, i, 0)),
            pl.BlockSpec((1, BM, D_FEAT), lambda i: (1, i, 0)),
            pl.BlockSpec((D_FEAT, D_FEAT), lambda i: (0, 0)),
            pl.BlockSpec((1, D_FEAT), lambda i: (0, 0)),
        ],
        out_specs=pl.BlockSpec((BM, D_FEAT), lambda i: (i, 0)),
        out_shape=jax.ShapeDtypeStruct((N_NODES, D_FEAT), jnp.float32),
    )(parts, parts, W, b2)


def kernel(feat, edge_index, W, b):
    lane = jnp.arange(PAD_N, dtype=jnp.int32) % PAD_ROWS
    pad = jnp.broadcast_to(
        jnp.stack([lane, N_NODES + lane])[:, None, :], (2, NW, PAD_N))
    idx = jnp.concatenate(
        [edge_index.astype(jnp.int32).reshape(2, NW, EPW_REAL), pad],
        axis=2).reshape(2, NW, NBLK, BLK)
    zeros = jnp.zeros((ROWS_PER_TILE, D_FEAT), jnp.float32)

    parts = _sc_gather_scatter(feat, idx, zeros)
    return _linear(parts, W, b.reshape(1, D_FEAT))


# speedup vs baseline: 1.0003x; 1.0003x over previous
import functools

import jax
import jax.numpy as jnp
from jax import lax
from jax.experimental import pallas as pl
from jax.experimental.pallas import tpu as pltpu
from jax.experimental.pallas import tpu_sc as plsc

N_NODES = 10000
N_EDGES = 320000
D_FEAT = 128

NC = 2
NS = 16
NW = NC * NS
BLK = 128
EPW_REAL = N_EDGES // NW
NBLK = 80
CH = 16
EPW = NBLK * BLK
PAD_N = EPW - EPW_REAL
PAD_ROWS = 16
ROWS_PER_TILE = 632
ACC_ROWS = NS * ROWS_PER_TILE


def _sc_gather_scatter(feat, idx, zeros):
    mesh = plsc.VectorSubcoreMesh(
        core_axis_name="c", subcore_axis_name="s",
        num_cores=NC, num_subcores=NS)

    @functools.partial(
        pl.kernel,
        out_type=jax.ShapeDtypeStruct((NC, ACC_ROWS, D_FEAT), jnp.float32),
        mesh=mesh,
        scratch_types=[
            pltpu.VMEM((CH, BLK), jnp.int32),
            pltpu.VMEM((CH, BLK), jnp.int32),
            pltpu.VMEM((2, BLK, D_FEAT), jnp.float32),
            pltpu.VMEM_SHARED((ACC_ROWS, D_FEAT), jnp.float32),
            pltpu.SemaphoreType.DMA,
            pltpu.SemaphoreType.DMA,
        ],
    )
    def k(feat_hbm, idx_hbm, zeros_hbm, out_hbm,
          src_v, dst_v, rows_v, accum, sem0, sem1):
        c = lax.axis_index("c")
        s = lax.axis_index("s")
        wid = s * NC + c
        row0 = s * ROWS_PER_TILE

        @pl.when(c == 0)
        def _():
            f0 = lax.min(row0, N_NODES - ROWS_PER_TILE)
            pltpu.sync_copy(feat_hbm.at[pl.ds(f0, ROWS_PER_TILE)],
                            accum.at[pl.ds(f0, ROWS_PER_TILE)])

        @pl.when(c == 1)
        def _():
            pltpu.sync_copy(zeros_hbm,
                            accum.at[pl.ds(row0, ROWS_PER_TILE)])

        plsc.subcore_barrier()

        sems = (sem0, sem1)

        def gather_start(j, b):
            pltpu.async_copy(feat_hbm.at[src_v.at[j]], rows_v.at[b], sems[b])

        def gather_wait(j, b):
            pltpu.make_async_copy(
                feat_hbm.at[src_v.at[j]], rows_v.at[b], sems[b]).wait()

        def chunk_body(ci, carry):
            pltpu.sync_copy(idx_hbm.at[0, wid, pl.ds(ci * CH, CH)], src_v)
            pltpu.sync_copy(idx_hbm.at[1, wid, pl.ds(ci * CH, CH)], dst_v)
            gather_start(0, 0)

            def pair_body(i, carry2):
                g = i * 2
                for b in range(2):
                    j = g + b
                    gather_start((j + 1) % CH, 1 - b)
                    gather_wait(j, b)
                    pltpu.sync_copy(rows_v.at[b], accum.at[dst_v.at[j]],
                                    add=True)
                return carry2

            lax.fori_loop(0, CH // 2, pair_body, 0)
            gather_wait(0, 0)
            return carry

        lax.fori_loop(0, NBLK // CH, chunk_body, 0)

        plsc.subcore_barrier()
        pltpu.sync_copy(accum.at[pl.ds(row0, ROWS_PER_TILE)],
                        out_hbm.at[c, pl.ds(row0, ROWS_PER_TILE)])

    return k(feat, idx, zeros)


def _linear(parts, W, b2):
    BM = 1000

    def mm_body(p0_ref, p1_ref, w_ref, b_ref, o_ref):
        h = p0_ref[0] + p1_ref[0]
        o_ref[...] = jnp.dot(h, w_ref[...],
                             preferred_element_type=jnp.float32) + b_ref[...]

    return pl.pallas_call(
        mm_body,
        grid=(N_NODES // BM,),
        in_specs=[
            pl.BlockSpec((1, BM, D_FEAT), lambda i: (0, i, 0)),
            pl.BlockSpec((1, BM, D_FEAT), lambda i: (1, i, 0)),
            pl.BlockSpec((D_FEAT, D_FEAT), lambda i: (0, 0)),
            pl.BlockSpec((1, D_FEAT), lambda i: (0, 0)),
        ],
        out_specs=pl.BlockSpec((BM, D_FEAT), lambda i: (i, 0)),
        out_shape=jax.ShapeDtypeStruct((N_NODES, D_FEAT), jnp.float32),
    )(parts, parts, W, b2)


def kernel(feat, edge_index, W, b):
    lane = jnp.arange(PAD_N, dtype=jnp.int32) % PAD_ROWS
    pad = jnp.broadcast_to(
        jnp.stack([lane, N_NODES + lane])[:, None, :], (2, NW, PAD_N))
    idx = jnp.concatenate(
        [edge_index.astype(jnp.int32).reshape(2, NW, EPW_REAL), pad],
        axis=2).reshape(2, NW, NBLK, BLK)
    zeros = jnp.zeros((ROWS_PER_TILE, D_FEAT), jnp.float32)

    parts = _sc_gather_scatter(feat, idx, zeros)
    return _linear(parts, W, b.reshape(1, D_FEAT))


# speedup vs baseline: 1.0051x; 1.0048x over previous
import functools

import jax
import jax.numpy as jnp
from jax import lax
from jax.experimental import pallas as pl
from jax.experimental.pallas import tpu as pltpu
from jax.experimental.pallas import tpu_sc as plsc

N_NODES = 10000
N_EDGES = 320000
D_FEAT = 128

NC = 2
NS = 16
NW = NC * NS
BLK = 128
EPW_REAL = N_EDGES // NW
NBLK = 80
CH = 16
EPW = NBLK * BLK
PAD_N = EPW - EPW_REAL
PAD_ROWS = 16
ROWS_PER_TILE = 632
ACC_ROWS = NS * ROWS_PER_TILE


def _sc_gather_scatter(feat, idx):
    mesh = plsc.VectorSubcoreMesh(
        core_axis_name="c", subcore_axis_name="s",
        num_cores=NC, num_subcores=NS)

    @functools.partial(
        pl.kernel,
        out_type=jax.ShapeDtypeStruct((NC, ACC_ROWS, D_FEAT), jnp.float32),
        mesh=mesh,
        scratch_types=[
            pltpu.VMEM((CH, BLK), jnp.int32),
            pltpu.VMEM((CH, BLK), jnp.int32),
            pltpu.VMEM((2, BLK, D_FEAT), jnp.float32),
            pltpu.VMEM_SHARED((ACC_ROWS, D_FEAT), jnp.float32),
            pltpu.SemaphoreType.DMA,
            pltpu.SemaphoreType.DMA,
        ],
    )
    def k(feat_hbm, idx_hbm, out_hbm,
          src_v, dst_v, rows_v, accum, sem0, sem1):
        c = lax.axis_index("c")
        s = lax.axis_index("s")
        wid = s * NC + c
        row0 = s * ROWS_PER_TILE

        @pl.when(c == 0)
        def _():
            f0 = lax.min(row0, N_NODES - ROWS_PER_TILE)
            pltpu.sync_copy(feat_hbm.at[pl.ds(f0, ROWS_PER_TILE)],
                            accum.at[pl.ds(f0, ROWS_PER_TILE)])

        @pl.when(c == 1)
        def _():
            zero = jnp.zeros((16,), jnp.float32)

            def zrow(i, cr):
                rows_v[0, i // 8, pl.ds((i % 8) * 16, 16)] = zero
                return cr

            lax.fori_loop(0, BLK * D_FEAT // 16, zrow, 0)
            for t in range(ROWS_PER_TILE // BLK):
                pltpu.sync_copy(rows_v.at[0],
                                accum.at[pl.ds(row0 + t * BLK, BLK)])
            rem = ROWS_PER_TILE % BLK
            pltpu.sync_copy(
                rows_v.at[0, pl.ds(0, rem)],
                accum.at[pl.ds(row0 + ROWS_PER_TILE - rem, rem)])

        plsc.subcore_barrier()

        sems = (sem0, sem1)

        def gather_start(j, b):
            pltpu.async_copy(feat_hbm.at[src_v.at[j]], rows_v.at[b], sems[b])

        def gather_wait(j, b):
            pltpu.make_async_copy(
                feat_hbm.at[src_v.at[j]], rows_v.at[b], sems[b]).wait()

        def chunk_body(ci, carry):
            pltpu.sync_copy(idx_hbm.at[0, wid, pl.ds(ci * CH, CH)], src_v)
            pltpu.sync_copy(idx_hbm.at[1, wid, pl.ds(ci * CH, CH)], dst_v)
            gather_start(0, 0)

            def pair_body(i, carry2):
                g = i * 2
                for b in range(2):
                    j = g + b
                    gather_start((j + 1) % CH, 1 - b)
                    gather_wait(j, b)
                    pltpu.sync_copy(rows_v.at[b], accum.at[dst_v.at[j]],
                                    add=True)
                return carry2

            lax.fori_loop(0, CH // 2, pair_body, 0)
            gather_wait(0, 0)
            return carry

        lax.fori_loop(0, NBLK // CH, chunk_body, 0)

        plsc.subcore_barrier()
        pltpu.sync_copy(accum.at[pl.ds(row0, ROWS_PER_TILE)],
                        out_hbm.at[c, pl.ds(row0, ROWS_PER_TILE)])

    return k(feat, idx)


def _linear(parts, W, b2):
    BM = 1000

    def mm_body(p0_ref, p1_ref, w_ref, b_ref, o_ref):
        h = p0_ref[0] + p1_ref[0]
        o_ref[...] = jnp.dot(h, w_ref[...],
                             preferred_element_type=jnp.float32) + b_ref[...]

    return pl.pallas_call(
        mm_body,
        grid=(N_NODES // BM,),
        in_specs=[
            pl.BlockSpec((1, BM, D_FEAT), lambda i: (0, i, 0)),
            pl.BlockSpec((1, BM, D_FEAT), lambda i: (1, i, 0)),
            pl.BlockSpec((D_FEAT, D_FEAT), lambda i: (0, 0)),
            pl.BlockSpec((1, D_FEAT), lambda i: (0, 0)),
        ],
        out_specs=pl.BlockSpec((BM, D_FEAT), lambda i: (i, 0)),
        out_shape=jax.ShapeDtypeStruct((N_NODES, D_FEAT), jnp.float32),
    )(parts, parts, W, b2)


def kernel(feat, edge_index, W, b):
    lane = jnp.arange(PAD_N, dtype=jnp.int32) % PAD_ROWS
    pad = jnp.broadcast_to(
        jnp.stack([lane, N_NODES + lane])[:, None, :], (2, NW, PAD_N))
    idx = jnp.concatenate(
        [edge_index.astype(jnp.int32).reshape(2, NW, EPW_REAL), pad],
        axis=2).reshape(2, NW, NBLK, BLK)

    parts = _sc_gather_scatter(feat, idx)
    return _linear(parts, W, b.reshape(1, D_FEAT))
